# baseline (speedup 1.0000x reference)
import jax
import jax.numpy as jnp
from jax.experimental import pallas as pl
from jax.experimental.pallas import tpu as pltpu

N = 10000
D = 128
BLK = 1000


def _tail_body(x0_ref, x1_ref, m0_ref, m1_ref,
               wr0_ref, br0_ref, wr1_ref, br1_ref, wl0_ref, wl1_ref,
               out0_ref, out1_ref):
    x0 = x0_ref[...]
    x1 = x1_ref[...]
    m0 = m0_ref[...]
    m1 = m1_ref[...]
    acc0 = jnp.dot(x0, wr0_ref[...].T, preferred_element_type=jnp.float32)
    acc0 = acc0 + jnp.dot(m1, wl1_ref[...].T, preferred_element_type=jnp.float32)
    out0_ref[...] = acc0 + br0_ref[...]
    acc1 = jnp.dot(x1, wr1_ref[...].T, preferred_element_type=jnp.float32)
    acc1 = acc1 + jnp.dot(m0, wl0_ref[...].T, preferred_element_type=jnp.float32)
    out1_ref[...] = acc1 + br1_ref[...]


def _tail(x0, x1, m0, m1, W_root0, b_root0, W_root1, b_root1, W_rel0, W_rel1):
    grid = (N // BLK,)
    blk = pl.BlockSpec((BLK, D), lambda i: (i, 0))
    full = pl.BlockSpec((D, D), lambda i: (0, 0))
    bias = pl.BlockSpec((1, D), lambda i: (0, 0))
    return pl.pallas_call(
        _tail_body,
        grid=grid,
        in_specs=[blk, blk, blk, blk, full, bias, full, bias, full, full],
        out_specs=[blk, blk],
        out_shape=[jax.ShapeDtypeStruct((N, D), jnp.float32),
                   jax.ShapeDtypeStruct((N, D), jnp.float32)],
    )(x0, x1, m0, m1, W_root0, b_root0.reshape(1, D), W_root1,
      b_root1.reshape(1, D), W_rel0, W_rel1)


def kernel(x0, x1, W_root0, b_root0, W_root1, b_root1, W_rel0, W_rel1,
           edge_index0, edge_index1):
    src0, dst0 = edge_index0[0], edge_index0[1]
    s0 = jnp.zeros((N, D), dtype=x0.dtype).at[dst0].add(x0[src0])
    c0 = jnp.zeros((N,), dtype=x0.dtype).at[dst0].add(1.0)
    m0 = s0 / jnp.maximum(c0, 1.0)[:, None]

    src1, dst1 = edge_index1[0], edge_index1[1]
    s1 = jnp.zeros((N, D), dtype=x1.dtype).at[dst1].add(x1[src1])
    c1 = jnp.zeros((N,), dtype=x1.dtype).at[dst1].add(1.0)
    m1 = s1 / jnp.maximum(c1, 1.0)[:, None]

    out0, out1 = _tail(x0, x1, m0, m1, W_root0, b_root0, W_root1, b_root1,
                       W_rel0, W_rel1)
    return (out0, out1)


# speedup vs baseline: 2.6150x; 2.6150x over previous
import jax
import jax.numpy as jnp
from jax import lax
from jax.experimental import pallas as pl
from jax.experimental.pallas import tpu as pltpu
from jax.experimental.pallas import tpu_sc as plsc

N = 10000
E = 320000
D = 128

CH = 128
NSUB = 16
NCHUNKS = E // CH
CPT = NCHUNKS // NSUB
XTRA = NCHUNKS - CPT * NSUB
NR = 10240
HALF = NR // 2
TRASH = HALF
SSH_R = HALF + 8
RPT_S = HALF // NSUB
RPT_C = NR // NSUB

BLK = 1000


def _agg_body(x_hbm, src_hbm, dst_hbm, zs_hbm, ones_hbm,
              s_out, c_out,
              ssh, csh, idx_s, idx_d, rowsA, ones_b, semA):
    c = lax.axis_index("c")
    w = lax.axis_index("s")

    pltpu.sync_copy(ones_hbm, ones_b)

    ebase = c * E
    xoff = jnp.full((16,), c * N, jnp.int32)

    for p in (0, 1):
        pltpu.sync_copy(zs_hbm, ssh.at[pl.ds(w * RPT_S, RPT_S)])
        pltpu.sync_copy(zs_hbm, csh.at[pl.ds(w * RPT_S, RPT_S)])
        plsc.subcore_barrier()

        def do_chunk(cn, p=p):
            off = ebase + cn * CH
            pltpu.sync_copy(src_hbm.at[pl.ds(off, CH)], idx_s)
            pltpu.sync_copy(dst_hbm.at[pl.ds(off, CH)], idx_d)
            for k in range(CH // 16):
                idx_s[pl.ds(16 * k, 16)] = idx_s[pl.ds(16 * k, 16)] + xoff
            for k in range(CH // 16):
                v = idx_d[pl.ds(16 * k, 16)]
                if p == 0:
                    nv = jnp.where(v < HALF, v, TRASH)
                else:
                    t = v - HALF
                    nv = jnp.where(t >= 0, t, TRASH)
                idx_d[pl.ds(16 * k, 16)] = nv
            pltpu.async_copy(x_hbm.at[idx_s], rowsA, semA).wait()
            pltpu.sync_copy(rowsA, ssh.at[idx_d], add=True)
            pltpu.sync_copy(ones_b, csh.at[idx_d], add=True)

        def chunk_body(t, _):
            do_chunk(w * CPT + t)
            return 0
        lax.fori_loop(0, CPT, chunk_body, 0)

        @pl.when(w < XTRA)
        def _():
            do_chunk(NSUB * CPT + w)

        plsc.subcore_barrier()
        pltpu.sync_copy(ssh.at[pl.ds(w * RPT_S, RPT_S)],
                        s_out.at[pl.ds(c * NR + p * HALF + w * RPT_S, RPT_S)])
        pltpu.sync_copy(csh.at[pl.ds(w * RPT_S, RPT_S)],
                        c_out.at[pl.ds(c * NR + p * HALF + w * RPT_S, RPT_S)])


def _aggregate(x0, x1, src0, dst0, src1, dst1):
    zs = jnp.zeros((RPT_S, D), jnp.float32)
    ones2d = jnp.ones((CH, D), jnp.float32)
    x_cat = jnp.concatenate([x0, x1], axis=0)
    src_cat = jnp.concatenate([src0, src1], axis=0)
    dst_cat = jnp.concatenate([dst0, dst1], axis=0)
    f = pl.kernel(
        _agg_body,
        mesh=plsc.VectorSubcoreMesh(core_axis_name="c", subcore_axis_name="s"),
        out_type=[
            jax.ShapeDtypeStruct((2 * NR, D), jnp.float32),
            jax.ShapeDtypeStruct((2 * NR, D), jnp.float32),
        ],
        scratch_types=[
            pltpu.VMEM_SHARED((SSH_R, D), jnp.float32),
            pltpu.VMEM_SHARED((SSH_R, D), jnp.float32),
            pltpu.VMEM((CH,), jnp.int32),
            pltpu.VMEM((CH,), jnp.int32),
            pltpu.VMEM((CH, D), jnp.float32),
            pltpu.VMEM((CH, D), jnp.float32),
            pltpu.SemaphoreType.DMA,
        ],
    )
    s_all, c_all = f(x_cat, src_cat, dst_cat, zs, ones2d)
    return (s_all[:N], c_all[:N, 0:1], s_all[NR:NR + N], c_all[NR:NR + N, 0:1])


def _tail_body(x0_ref, x1_ref, s0_ref, c0_ref, s1_ref, c1_ref,
               wr0_ref, br0_ref, wr1_ref, br1_ref, wl0_ref, wl1_ref,
               out0_ref, out1_ref):
    m0 = s0_ref[...] / jnp.maximum(c0_ref[...], 1.0)
    m1 = s1_ref[...] / jnp.maximum(c1_ref[...], 1.0)
    acc0 = jnp.dot(x0_ref[...], wr0_ref[...].T, preferred_element_type=jnp.float32)
    acc0 = acc0 + jnp.dot(m1, wl1_ref[...].T, preferred_element_type=jnp.float32)
    out0_ref[...] = acc0 + br0_ref[...]
    acc1 = jnp.dot(x1_ref[...], wr1_ref[...].T, preferred_element_type=jnp.float32)
    acc1 = acc1 + jnp.dot(m0, wl0_ref[...].T, preferred_element_type=jnp.float32)
    out1_ref[...] = acc1 + br1_ref[...]


def _tail(x0, x1, s0, c0, s1, c1,
          W_root0, b_root0, W_root1, b_root1, W_rel0, W_rel1):
    grid = (N // BLK,)
    blk = pl.BlockSpec((BLK, D), lambda i: (i, 0))
    cblk = pl.BlockSpec((BLK, 1), lambda i: (i, 0))
    full = pl.BlockSpec((D, D), lambda i: (0, 0))
    bias = pl.BlockSpec((1, D), lambda i: (0, 0))
    return pl.pallas_call(
        _tail_body,
        grid=grid,
        in_specs=[blk, blk, blk, cblk, blk, cblk,
                  full, bias, full, bias, full, full],
        out_specs=[blk, blk],
        out_shape=[jax.ShapeDtypeStruct((N, D), jnp.float32),
                   jax.ShapeDtypeStruct((N, D), jnp.float32)],
    )(x0, x1, s0, c0, s1, c1, W_root0, b_root0.reshape(1, D),
      W_root1, b_root1.reshape(1, D), W_rel0, W_rel1)


def kernel(x0, x1, W_root0, b_root0, W_root1, b_root1, W_rel0, W_rel1,
           edge_index0, edge_index1):
    src0, dst0 = edge_index0[0], edge_index0[1]
    src1, dst1 = edge_index1[0], edge_index1[1]
    s0, c0, s1, c1 = _aggregate(x0, x1, src0, dst0, src1, dst1)
    out0, out1 = _tail(x0, x1, s0, c0, s1, c1, W_root0, b_root0,
                       W_root1, b_root1, W_rel0, W_rel1)
    return (out0, out1)


# speedup vs baseline: 2.9623x; 1.1328x over previous
import jax
import jax.numpy as jnp
from jax import lax
from jax.experimental import pallas as pl
from jax.experimental.pallas import tpu as pltpu
from jax.experimental.pallas import tpu_sc as plsc

N = 10000
E = 320000
D = 128

CH = 64
NSUB = 16
NCHUNKS = E // CH
CPT = NCHUNKS // NSUB
XTRA = NCHUNKS - CPT * NSUB
NR = 10240
HALF = NR // 2
TRASH = HALF
SSH_R = HALF + 8
RPT_S = HALF // NSUB
RPT_C = NR // NSUB

BLK = 1000


def _agg_body(x_hbm, src_hbm, dst_hbm, zs_hbm, ones_hbm,
              s_out, c_out,
              ssh, csh, idx_sA, idx_dA, rowsA, idx_sB, idx_dB, rowsB,
              ones_b, semA, semB):
    c = lax.axis_index("c")
    w = lax.axis_index("s")

    pltpu.sync_copy(ones_hbm, ones_b)

    ebase = c * E
    xoff = jnp.full((16,), c * N, jnp.int32)

    for p in (0, 1):
        pltpu.sync_copy(zs_hbm, ssh.at[pl.ds(w * RPT_S, RPT_S)])
        pltpu.sync_copy(zs_hbm, csh.at[pl.ds(w * RPT_S, RPT_S)])
        plsc.subcore_barrier()

        def fire(cn, idx_s, idx_d, rows, sem):
            off = ebase + cn * CH
            pltpu.sync_copy(src_hbm.at[pl.ds(off, CH)], idx_s)
            pltpu.sync_copy(dst_hbm.at[pl.ds(off, CH)], idx_d)
            for k in range(CH // 16):
                idx_s[pl.ds(16 * k, 16)] = idx_s[pl.ds(16 * k, 16)] + xoff
            pltpu.async_copy(x_hbm.at[idx_s], rows, sem)

        def drain(idx_s, idx_d, rows, sem, p=p):
            pltpu.make_async_copy(x_hbm.at[idx_s], rows, sem).wait()
            for k in range(CH // 16):
                v = idx_d[pl.ds(16 * k, 16)]
                if p == 0:
                    nv = jnp.where(v < HALF, v, TRASH)
                else:
                    t = v - HALF
                    nv = jnp.where(t >= 0, t, TRASH)
                idx_d[pl.ds(16 * k, 16)] = nv
            pltpu.sync_copy(rows, ssh.at[idx_d], add=True)
            pltpu.sync_copy(ones_b, csh.at[idx_d], add=True)

        base = w * CPT
        fire(base, idx_sA, idx_dA, rowsA, semA)

        def pair_body(t, _):
            cn = base + 2 * t
            fire(cn + 1, idx_sB, idx_dB, rowsB, semB)
            drain(idx_sA, idx_dA, rowsA, semA)

            @pl.when(cn + 2 < base + CPT)
            def _():
                fire(cn + 2, idx_sA, idx_dA, rowsA, semA)
            drain(idx_sB, idx_dB, rowsB, semB)
            return 0
        lax.fori_loop(0, CPT // 2, pair_body, 0)

        @pl.when(w < XTRA)
        def _():
            fire(NSUB * CPT + w, idx_sA, idx_dA, rowsA, semA)
            drain(idx_sA, idx_dA, rowsA, semA)

        plsc.subcore_barrier()
        pltpu.sync_copy(ssh.at[pl.ds(w * RPT_S, RPT_S)],
                        s_out.at[pl.ds(c * NR + p * HALF + w * RPT_S, RPT_S)])
        pltpu.sync_copy(csh.at[pl.ds(w * RPT_S, RPT_S)],
                        c_out.at[pl.ds(c * NR + p * HALF + w * RPT_S, RPT_S)])


def _aggregate(x0, x1, src0, dst0, src1, dst1):
    zs = jnp.zeros((RPT_S, D), jnp.float32)
    ones2d = jnp.ones((CH, D), jnp.float32)
    x_cat = jnp.concatenate([x0, x1], axis=0)
    src_cat = jnp.concatenate([src0, src1], axis=0)
    dst_cat = jnp.concatenate([dst0, dst1], axis=0)
    f = pl.kernel(
        _agg_body,
        mesh=plsc.VectorSubcoreMesh(core_axis_name="c", subcore_axis_name="s"),
        out_type=[
            jax.ShapeDtypeStruct((2 * NR, D), jnp.float32),
            jax.ShapeDtypeStruct((2 * NR, D), jnp.float32),
        ],
        scratch_types=[
            pltpu.VMEM_SHARED((SSH_R, D), jnp.float32),
            pltpu.VMEM_SHARED((SSH_R, D), jnp.float32),
            pltpu.VMEM((CH,), jnp.int32),
            pltpu.VMEM((CH,), jnp.int32),
            pltpu.VMEM((CH, D), jnp.float32),
            pltpu.VMEM((CH,), jnp.int32),
            pltpu.VMEM((CH,), jnp.int32),
            pltpu.VMEM((CH, D), jnp.float32),
            pltpu.VMEM((CH, D), jnp.float32),
            pltpu.SemaphoreType.DMA,
            pltpu.SemaphoreType.DMA,
        ],
    )
    s_all, c_all = f(x_cat, src_cat, dst_cat, zs, ones2d)
    return (s_all[:N], c_all[:N, 0:1], s_all[NR:NR + N], c_all[NR:NR + N, 0:1])


def _tail_body(x0_ref, x1_ref, s0_ref, c0_ref, s1_ref, c1_ref,
               wr0_ref, br0_ref, wr1_ref, br1_ref, wl0_ref, wl1_ref,
               out0_ref, out1_ref):
    m0 = s0_ref[...] / jnp.maximum(c0_ref[...], 1.0)
    m1 = s1_ref[...] / jnp.maximum(c1_ref[...], 1.0)
    acc0 = jnp.dot(x0_ref[...], wr0_ref[...].T, preferred_element_type=jnp.float32)
    acc0 = acc0 + jnp.dot(m1, wl1_ref[...].T, preferred_element_type=jnp.float32)
    out0_ref[...] = acc0 + br0_ref[...]
    acc1 = jnp.dot(x1_ref[...], wr1_ref[...].T, preferred_element_type=jnp.float32)
    acc1 = acc1 + jnp.dot(m0, wl0_ref[...].T, preferred_element_type=jnp.float32)
    out1_ref[...] = acc1 + br1_ref[...]


def _tail(x0, x1, s0, c0, s1, c1,
          W_root0, b_root0, W_root1, b_root1, W_rel0, W_rel1):
    grid = (N // BLK,)
    blk = pl.BlockSpec((BLK, D), lambda i: (i, 0))
    cblk = pl.BlockSpec((BLK, 1), lambda i: (i, 0))
    full = pl.BlockSpec((D, D), lambda i: (0, 0))
    bias = pl.BlockSpec((1, D), lambda i: (0, 0))
    return pl.pallas_call(
        _tail_body,
        grid=grid,
        in_specs=[blk, blk, blk, cblk, blk, cblk,
                  full, bias, full, bias, full, full],
        out_specs=[blk, blk],
        out_shape=[jax.ShapeDtypeStruct((N, D), jnp.float32),
                   jax.ShapeDtypeStruct((N, D), jnp.float32)],
    )(x0, x1, s0, c0, s1, c1, W_root0, b_root0.reshape(1, D),
      W_root1, b_root1.reshape(1, D), W_rel0, W_rel1)


def kernel(x0, x1, W_root0, b_root0, W_root1, b_root1, W_rel0, W_rel1,
           edge_index0, edge_index1):
    src0, dst0 = edge_index0[0], edge_index0[1]
    src1, dst1 = edge_index1[0], edge_index1[1]
    s0, c0, s1, c1 = _aggregate(x0, x1, src0, dst0, src1, dst1)
    out0, out1 = _tail(x0, x1, s0, c0, s1, c1, W_root0, b_root0,
                       W_root1, b_root1, W_rel0, W_rel1)
    return (out0, out1)


# speedup vs baseline: 3.1230x; 1.0543x over previous
import jax
import jax.numpy as jnp
from jax import lax
from jax.experimental import pallas as pl
from jax.experimental.pallas import tpu as pltpu
from jax.experimental.pallas import tpu_sc as plsc

N = 10000
E = 320000
D = 128

CH = 80
NSUB = 16
NCHUNKS = E // CH
CPT = NCHUNKS // NSUB
XTRA = NCHUNKS - CPT * NSUB
NR = 10240
HALF = NR // 2
TRASH = HALF
SSH_R = HALF + 8
RPT_S = HALF // NSUB
RPT_C = NR // NSUB

BLK = 1000


def _agg_body(x_hbm, src_hbm, dst_hbm, zs_hbm, ones_hbm,
              s_out, c_out,
              ssh, csh, idx_sA, idx_dA, rowsA, idx_sB, idx_dB, rowsB,
              ones_b, semA, semB):
    c = lax.axis_index("c")
    w = lax.axis_index("s")

    pltpu.sync_copy(ones_hbm, ones_b)

    ebase = c * E
    xoff = jnp.full((16,), c * N, jnp.int32)

    for p in (0, 1):
        pltpu.sync_copy(zs_hbm, ssh.at[pl.ds(w * RPT_S, RPT_S)])
        pltpu.sync_copy(zs_hbm, csh.at[pl.ds(w * RPT_S, RPT_S)])
        plsc.subcore_barrier()

        def fire(cn, idx_s, idx_d, rows, sem):
            off = ebase + cn * CH
            pltpu.sync_copy(src_hbm.at[pl.ds(off, CH)], idx_s)
            pltpu.sync_copy(dst_hbm.at[pl.ds(off, CH)], idx_d)
            for k in range(CH // 16):
                idx_s[pl.ds(16 * k, 16)] = idx_s[pl.ds(16 * k, 16)] + xoff
            pltpu.async_copy(x_hbm.at[idx_s], rows, sem)

        def drain(idx_s, idx_d, rows, sem, p=p):
            pltpu.make_async_copy(x_hbm.at[idx_s], rows, sem).wait()
            for k in range(CH // 16):
                v = idx_d[pl.ds(16 * k, 16)]
                if p == 0:
                    nv = jnp.where(v < HALF, v, TRASH)
                else:
                    t = v - HALF
                    nv = jnp.where(t >= 0, t, TRASH)
                idx_d[pl.ds(16 * k, 16)] = nv
            pltpu.sync_copy(rows, ssh.at[idx_d], add=True)
            pltpu.sync_copy(ones_b, csh.at[idx_d], add=True)

        base = w * CPT
        fire(base, idx_sA, idx_dA, rowsA, semA)

        def pair_body(t, _):
            cn = base + 2 * t
            fire(cn + 1, idx_sB, idx_dB, rowsB, semB)
            drain(idx_sA, idx_dA, rowsA, semA)

            @pl.when(cn + 2 < base + CPT)
            def _():
                fire(cn + 2, idx_sA, idx_dA, rowsA, semA)
            drain(idx_sB, idx_dB, rowsB, semB)
            return 0
        lax.fori_loop(0, CPT // 2, pair_body, 0)

        @pl.when(w < XTRA)
        def _():
            fire(NSUB * CPT + w, idx_sA, idx_dA, rowsA, semA)
            drain(idx_sA, idx_dA, rowsA, semA)

        plsc.subcore_barrier()
        pltpu.sync_copy(ssh.at[pl.ds(w * RPT_S, RPT_S)],
                        s_out.at[pl.ds(c * NR + p * HALF + w * RPT_S, RPT_S)])
        pltpu.sync_copy(csh.at[pl.ds(w * RPT_S, RPT_S)],
                        c_out.at[pl.ds(c * NR + p * HALF + w * RPT_S, RPT_S)])


def _aggregate(x0, x1, src0, dst0, src1, dst1):
    zs = jnp.zeros((RPT_S, D), jnp.float32)
    ones2d = jnp.ones((CH, D), jnp.float32)
    x_cat = jnp.concatenate([x0, x1], axis=0)
    src_cat = jnp.concatenate([src0, src1], axis=0)
    dst_cat = jnp.concatenate([dst0, dst1], axis=0)
    f = pl.kernel(
        _agg_body,
        mesh=plsc.VectorSubcoreMesh(core_axis_name="c", subcore_axis_name="s"),
        out_type=[
            jax.ShapeDtypeStruct((2 * NR, D), jnp.float32),
            jax.ShapeDtypeStruct((2 * NR, D), jnp.float32),
        ],
        scratch_types=[
            pltpu.VMEM_SHARED((SSH_R, D), jnp.float32),
            pltpu.VMEM_SHARED((SSH_R, D), jnp.float32),
            pltpu.VMEM((CH,), jnp.int32),
            pltpu.VMEM((CH,), jnp.int32),
            pltpu.VMEM((CH, D), jnp.float32),
            pltpu.VMEM((CH,), jnp.int32),
            pltpu.VMEM((CH,), jnp.int32),
            pltpu.VMEM((CH, D), jnp.float32),
            pltpu.VMEM((CH, D), jnp.float32),
            pltpu.SemaphoreType.DMA,
            pltpu.SemaphoreType.DMA,
        ],
    )
    s_all, c_all = f(x_cat, src_cat, dst_cat, zs, ones2d)
    return (s_all[:N], c_all[:N, 0:1], s_all[NR:NR + N], c_all[NR:NR + N, 0:1])


def _tail_body(x0_ref, x1_ref, s0_ref, c0_ref, s1_ref, c1_ref,
               wr0_ref, br0_ref, wr1_ref, br1_ref, wl0_ref, wl1_ref,
               out0_ref, out1_ref):
    m0 = s0_ref[...] / jnp.maximum(c0_ref[...], 1.0)
    m1 = s1_ref[...] / jnp.maximum(c1_ref[...], 1.0)
    acc0 = jnp.dot(x0_ref[...], wr0_ref[...].T, preferred_element_type=jnp.float32)
    acc0 = acc0 + jnp.dot(m1, wl1_ref[...].T, preferred_element_type=jnp.float32)
    out0_ref[...] = acc0 + br0_ref[...]
    acc1 = jnp.dot(x1_ref[...], wr1_ref[...].T, preferred_element_type=jnp.float32)
    acc1 = acc1 + jnp.dot(m0, wl0_ref[...].T, preferred_element_type=jnp.float32)
    out1_ref[...] = acc1 + br1_ref[...]


def _tail(x0, x1, s0, c0, s1, c1,
          W_root0, b_root0, W_root1, b_root1, W_rel0, W_rel1):
    grid = (N // BLK,)
    blk = pl.BlockSpec((BLK, D), lambda i: (i, 0))
    cblk = pl.BlockSpec((BLK, 1), lambda i: (i, 0))
    full = pl.BlockSpec((D, D), lambda i: (0, 0))
    bias = pl.BlockSpec((1, D), lambda i: (0, 0))
    return pl.pallas_call(
        _tail_body,
        grid=grid,
        in_specs=[blk, blk, blk, cblk, blk, cblk,
                  full, bias, full, bias, full, full],
        out_specs=[blk, blk],
        out_shape=[jax.ShapeDtypeStruct((N, D), jnp.float32),
                   jax.ShapeDtypeStruct((N, D), jnp.float32)],
    )(x0, x1, s0, c0, s1, c1, W_root0, b_root0.reshape(1, D),
      W_root1, b_root1.reshape(1, D), W_rel0, W_rel1)


def kernel(x0, x1, W_root0, b_root0, W_root1, b_root1, W_rel0, W_rel1,
           edge_index0, edge_index1):
    src0, dst0 = edge_index0[0], edge_index0[1]
    src1, dst1 = edge_index1[0], edge_index1[1]
    s0, c0, s1, c1 = _aggregate(x0, x1, src0, dst0, src1, dst1)
    out0, out1 = _tail(x0, x1, s0, c0, s1, c1, W_root0, b_root0,
                       W_root1, b_root1, W_rel0, W_rel1)
    return (out0, out1)
